# baseline (speedup 1.0000x reference)
import functools

import jax
import jax.numpy as jnp
from jax import lax
from jax.experimental import pallas as pl
from jax.experimental.pallas import tpu as pltpu
from jax.experimental.pallas import tpu_sc as plsc

EMBED_DIM = 64
ROWS_PER_CHUNK = 8
NUM_WORKERS = 32


def _emb_body(z, table, out, idx_v, rows_v, sem_i, sem_g, sem_w):
    n_batch, seq = z.shape
    per_w = n_batch // NUM_WORKERS
    n_chunks = per_w // ROWS_PER_CHUNK
    wid = lax.axis_index("s") * 2 + lax.axis_index("c")
    base = wid * per_w

    pltpu.async_copy(z.at[pl.ds(base, ROWS_PER_CHUNK)], idx_v.at[0],
                     sem_i.at[0])

    def pair(i, carry):
        for b in range(2):
            ci = 2 * i + b
            r0 = base + ci * ROWS_PER_CHUNK
            @pl.when(ci + 1 < n_chunks)
            def _():
                pltpu.async_copy(
                    z.at[pl.ds(r0 + ROWS_PER_CHUNK, ROWS_PER_CHUNK)],
                    idx_v.at[1 - b], sem_i.at[1 - b])
            pltpu.make_async_copy(
                z.at[pl.ds(r0, ROWS_PER_CHUNK)], idx_v.at[b],
                sem_i.at[b]).wait()
            @pl.when(ci >= 2)
            def _():
                pltpu.make_async_copy(
                    rows_v.at[b], out.at[pl.ds(r0, ROWS_PER_CHUNK)],
                    sem_w.at[b]).wait()
            copies = [
                pltpu.async_copy(table.at[idx_v.at[b, j]], rows_v.at[b, j],
                                 sem_g.at[b])
                for j in range(ROWS_PER_CHUNK)
            ]
            for c in copies:
                c.wait()
            pltpu.async_copy(rows_v.at[b], out.at[pl.ds(r0, ROWS_PER_CHUNK)],
                             sem_w.at[b])
        return carry

    lax.fori_loop(0, n_chunks // 2, pair, 0)

    for b in range(2):
        r0 = base + (n_chunks - 2 + b) * ROWS_PER_CHUNK
        pltpu.make_async_copy(
            rows_v.at[b], out.at[pl.ds(r0, ROWS_PER_CHUNK)],
            sem_w.at[b]).wait()


@jax.jit
def kernel(z, table):
    B, S = z.shape
    z = z.astype(jnp.int32)
    table = table.at[0].set(jnp.zeros((table.shape[1],), table.dtype))

    mesh = plsc.VectorSubcoreMesh(core_axis_name="c", subcore_axis_name="s")
    out = pl.kernel(
        _emb_body,
        mesh=mesh,
        out_type=jax.ShapeDtypeStruct((B, S, EMBED_DIM), jnp.float32),
        scratch_types=[
            pltpu.VMEM((2, ROWS_PER_CHUNK, 50), jnp.int32),
            pltpu.VMEM((2, ROWS_PER_CHUNK, 50, EMBED_DIM), jnp.float32),
            pltpu.SemaphoreType.DMA((2,)),
            pltpu.SemaphoreType.DMA((2,)),
            pltpu.SemaphoreType.DMA((2,)),
        ],
        compiler_params=pltpu.CompilerParams(use_tc_tiling_on_sc=False),
    )(z, table)
    return out
